# Initial kernel scaffold; baseline (speedup 1.0000x reference)
import jax
import jax.numpy as jnp
from jax.experimental import pallas as pl


def kernel(x, graph_indices, W1, b1, W2, b2, Wa, ba, Wo1, bo1, Wo2, bo2):
    raise NotImplementedError("write your pallas kernel here")


# speedup vs baseline: 1.0175x; 1.0175x over previous
import jax
import jax.numpy as jnp
from jax.experimental import pallas as pl

N = 10000
D = 128
G = 2
ROW_BLK = 1000


def _dense1_body(x_ref, W1_ref, b1_ref, W2_ref, b2_ref, Wa_ref, ba_ref,
                 h_ref, a_ref):
    x = x_ref[...]
    h1 = jnp.maximum(jnp.dot(x, W1_ref[...],
                             preferred_element_type=jnp.float32) + b1_ref[...], 0.0)
    h = jnp.maximum(jnp.dot(h1, W2_ref[...],
                            preferred_element_type=jnp.float32) + b2_ref[...], 0.0)
    h_ref[...] = h
    a_ref[...] = jnp.dot(h, Wa_ref[...],
                         preferred_element_type=jnp.float32) + ba_ref[...]


def _dense2_body(cat_ref, Wo1_ref, bo1_ref, Wo2_ref, bo2_ref, out_ref):
    t = jnp.maximum(jnp.dot(cat_ref[...], Wo1_ref[...],
                            preferred_element_type=jnp.float32) + bo1_ref[...], 0.0)
    out_ref[...] = jnp.dot(t, Wo2_ref[...],
                           preferred_element_type=jnp.float32) + bo2_ref[...]


def _dense1(x, W1, b1, W2, b2, Wa, ba):
    grid = (N // ROW_BLK,)
    return pl.pallas_call(
        _dense1_body,
        grid=grid,
        in_specs=[
            pl.BlockSpec((ROW_BLK, D), lambda i: (i, 0)),
            pl.BlockSpec((D, D), lambda i: (0, 0)),
            pl.BlockSpec((1, D), lambda i: (0, 0)),
            pl.BlockSpec((D, D), lambda i: (0, 0)),
            pl.BlockSpec((1, D), lambda i: (0, 0)),
            pl.BlockSpec((D, 3 * G * D), lambda i: (0, 0)),
            pl.BlockSpec((1, 3 * G * D), lambda i: (0, 0)),
        ],
        out_specs=[
            pl.BlockSpec((ROW_BLK, D), lambda i: (i, 0)),
            pl.BlockSpec((ROW_BLK, 3 * G * D), lambda i: (i, 0)),
        ],
        out_shape=[
            jax.ShapeDtypeStruct((N, D), jnp.float32),
            jax.ShapeDtypeStruct((N, 3 * G * D), jnp.float32),
        ],
    )(x, W1, b1.reshape(1, D), W2, b2.reshape(1, D), Wa,
      ba.reshape(1, 3 * G * D))


def _dense2(cat, Wo1, bo1, Wo2, bo2):
    grid = (N // ROW_BLK,)
    return pl.pallas_call(
        _dense2_body,
        grid=grid,
        in_specs=[
            pl.BlockSpec((ROW_BLK, (1 + G) * D), lambda i: (i, 0)),
            pl.BlockSpec(((1 + G) * D, D), lambda i: (0, 0)),
            pl.BlockSpec((1, D), lambda i: (0, 0)),
            pl.BlockSpec((D, D), lambda i: (0, 0)),
            pl.BlockSpec((1, D), lambda i: (0, 0)),
        ],
        out_specs=pl.BlockSpec((ROW_BLK, D), lambda i: (i, 0)),
        out_shape=jax.ShapeDtypeStruct((N, D), jnp.float32),
    )(cat, Wo1, bo1.reshape(1, D), Wo2, bo2.reshape(1, D))


def kernel(x, graph_indices, W1, b1, W2, b2, Wa, ba, Wo1, bo1, Wo2, bo2):
    h, a = _dense1(x, W1, b1, W2, b2, Wa, ba)
    a4 = a.reshape(N, G, 3, D)
    q = a4[:, :, 0, :]
    k = a4[:, :, 1, :]
    v = a4[:, :, 2, :]
    x2_list = []
    for i in range(G):
        src = graph_indices[i, 0]
        dst = graph_indices[i, 1]
        q_i = q[src, i, :]
        k_i = k[dst, i, :]
        score = jnp.sum(q_i * k_i, axis=-1) / jnp.sqrt(float(D))
        m = jax.ops.segment_max(score, src, num_segments=N)
        e = jnp.exp(score - m[src])
        denom = jax.ops.segment_sum(e, src, num_segments=N)
        attn = e / denom[src]
        x2 = jax.ops.segment_sum(attn[:, None] * v[dst, i, :], src,
                                 num_segments=N)
        x2_list.append(x2)
    cat = jnp.concatenate([h] + x2_list, axis=-1)
    return _dense2(cat, Wo1, bo1, Wo2, bo2)


# speedup vs baseline: 9.4078x; 9.2458x over previous
import functools

import jax
import jax.numpy as jnp
from jax import lax
from jax.experimental import pallas as pl
from jax.experimental.pallas import tpu as pltpu
from jax.experimental.pallas import tpu_sc as plsc

N = 10000
D = 128
G = 2
E = 320000
ROW_BLK = 1000

NC = 2
NS = 16
L = 16

ROWS_BASE = 624
HALF_BASE = 312
TRASH = 320

CE = 2000
NCHUNK = E // CE
CG = 32
CV = 64
CAP = 11264

INV_SQRT_D = 1.0 / float(D) ** 0.5
NEG_BIG = -3.0e38


def _dense1_body(x_ref, W1_ref, b1_ref, W2_ref, b2_ref, Wa_ref, ba_ref,
                 h_ref, a_ref):
    x = x_ref[...]
    h1 = jnp.maximum(jnp.dot(x, W1_ref[...],
                             preferred_element_type=jnp.float32) + b1_ref[...], 0.0)
    h = jnp.maximum(jnp.dot(h1, W2_ref[...],
                            preferred_element_type=jnp.float32) + b2_ref[...], 0.0)
    h_ref[...] = h
    a_ref[...] = jnp.dot(h, Wa_ref[...],
                         preferred_element_type=jnp.float32) + ba_ref[...]


def _dense2_body(cat_ref, Wo1_ref, bo1_ref, Wo2_ref, bo2_ref, out_ref):
    t = jnp.maximum(jnp.dot(cat_ref[...], Wo1_ref[...],
                            preferred_element_type=jnp.float32) + bo1_ref[...], 0.0)
    out_ref[...] = jnp.dot(t, Wo2_ref[...],
                           preferred_element_type=jnp.float32) + bo2_ref[...]


def _dense1(x, W1, b1, W2, b2, Wa, ba):
    return pl.pallas_call(
        _dense1_body,
        grid=(N // ROW_BLK,),
        in_specs=[
            pl.BlockSpec((ROW_BLK, D), lambda i: (i, 0)),
            pl.BlockSpec((D, D), lambda i: (0, 0)),
            pl.BlockSpec((1, D), lambda i: (0, 0)),
            pl.BlockSpec((D, D), lambda i: (0, 0)),
            pl.BlockSpec((1, D), lambda i: (0, 0)),
            pl.BlockSpec((D, 3 * G * D), lambda i: (0, 0)),
            pl.BlockSpec((1, 3 * G * D), lambda i: (0, 0)),
        ],
        out_specs=[
            pl.BlockSpec((ROW_BLK, D), lambda i: (i, 0)),
            pl.BlockSpec((ROW_BLK, 3 * G * D), lambda i: (i, 0)),
        ],
        out_shape=[
            jax.ShapeDtypeStruct((N, D), jnp.float32),
            jax.ShapeDtypeStruct((N, 3 * G * D), jnp.float32),
        ],
    )(x, W1, b1.reshape(1, D), W2, b2.reshape(1, D), Wa,
      ba.reshape(1, 3 * G * D))


def _dense2(cat, Wo1, bo1, Wo2, bo2):
    return pl.pallas_call(
        _dense2_body,
        grid=(N // ROW_BLK,),
        in_specs=[
            pl.BlockSpec((ROW_BLK, (1 + G) * D), lambda i: (i, 0)),
            pl.BlockSpec(((1 + G) * D, D), lambda i: (0, 0)),
            pl.BlockSpec((1, D), lambda i: (0, 0)),
            pl.BlockSpec((D, D), lambda i: (0, 0)),
            pl.BlockSpec((1, D), lambda i: (0, 0)),
        ],
        out_specs=pl.BlockSpec((ROW_BLK, D), lambda i: (i, 0)),
        out_shape=jax.ShapeDtypeStruct((N, D), jnp.float32),
    )(cat, Wo1, bo1.reshape(1, D), Wo2, bo2.reshape(1, D))


def _splat_i32(s):
    return jnp.full((L,), s, jnp.int32)


def _splat_f32(s):
    return jnp.full((L,), s, jnp.float32)


def _sc_body(q0, k0, v0, src0, dst0, q1, k1, v1, src1, dst1,
             x2_0, x2_1,
             stage_src, stage_dst,
             selsA, seldA, ebufA, selsB, seldB, ebufB,
             gbuf, x2buf, mbuf, dbuf, sem):
    cid = lax.axis_index("c")
    sid = lax.axis_index("s")
    last = sid == NS - 1
    lo = sid * ROWS_BASE
    h0 = jnp.where(last, 320, HALF_BASE).astype(jnp.int32)
    upper = jnp.where(last, 640, ROWS_BASE).astype(jnp.int32)
    lane_iota = lax.iota(jnp.int32, L)
    lane0 = lane_iota == 0

    def pipeline(q_hbm, k_hbm, v_hbm, src_hbm, dst_hbm, out_hbm):
        start = (sid * (NCHUNK // NS)).astype(jnp.int32)

        def chunk_body(cc, cnts):
            chunk = lax.rem(start + cc, NCHUNK)
            off = chunk * CE
            pltpu.async_copy(src_hbm.at[pl.ds(off, CE)], stage_src, sem).wait()
            pltpu.async_copy(dst_hbm.at[pl.ds(off, CE)], stage_dst, sem).wait()

            def grp_body(j, cnts):
                cntA, cntB = cnts
                sv = stage_src[pl.ds(j * L, L)]
                dv = stage_dst[pl.ds(j * L, L)]
                rel = sv - lo
                mA = (rel >= 0) & (rel < h0)
                mB = (rel >= h0) & (rel < upper)
                plsc.store_compressed(selsA.at[pl.ds(cntA, L)], sv, mask=mA)
                plsc.store_compressed(seldA.at[pl.ds(cntA, L)], dv, mask=mA)
                plsc.store_compressed(selsB.at[pl.ds(cntB, L)], sv, mask=mB)
                plsc.store_compressed(seldB.at[pl.ds(cntB, L)], dv, mask=mB)
                pA = plsc.all_reduce_population_count(mA)
                pB = plsc.all_reduce_population_count(mB)
                return (cntA + pA[0], cntB + pB[0])

            return lax.fori_loop(0, CE // L, grp_body, cnts)

        cntA, cntB = lax.fori_loop(0, NCHUNK, chunk_body,
                                   (jnp.int32(0), jnp.int32(0)))

        for half, (sels, seld, ebuf, cnt, rbase) in enumerate([
                (selsA, seldA, ebufA, cntA, lo),
                (selsB, seldB, ebufB, cntB, lo + h0)]):
            pad_src = rbase + TRASH
            for t in range(CV // L):
                sels[pl.ds(cnt + t * L, L)] = _splat_i32(pad_src)
                seld[pl.ds(cnt + t * L, L)] = _splat_i32(0)
            cnt_pad = ((cnt + CV - 1) // CV) * CV

            def init_md(j, _):
                mbuf[pl.ds(j * L, L)] = _splat_f32(NEG_BIG)
                dbuf[pl.ds(j * L, L)] = _splat_f32(0.0)
                return 0
            lax.fori_loop(0, (TRASH + L) // L, init_md, 0)

            def init_x2(j, _):
                for kk in range(D // L):
                    x2buf[j, pl.ds(kk * L, L)] = _splat_f32(0.0)
                return 0
            lax.fori_loop(0, TRASH + 1, init_x2, 0)

            def score_grp(p, _):
                gb = p * CG
                pltpu.async_copy(q_hbm.at[sels.at[pl.ds(gb, CG)]],
                                 gbuf.at[pl.ds(0, CG)], sem).wait()
                pltpu.async_copy(k_hbm.at[seld.at[pl.ds(gb, CG)]],
                                 gbuf.at[pl.ds(CG, CG)], sem).wait()

                def edge_body(i, _):
                    acc = gbuf[i, pl.ds(0, L)] * gbuf[CG + i, pl.ds(0, L)]
                    for kk in range(1, D // L):
                        acc = acc + (gbuf[i, pl.ds(kk * L, L)] *
                                     gbuf[CG + i, pl.ds(kk * L, L)])
                    s = jnp.sum(acc, axis=0) * INV_SQRT_D
                    ssp = _splat_f32(s)
                    plsc.store_scatter(ebuf, [_splat_i32(gb + i)], ssp,
                                       mask=lane0)
                    sv = sels[pl.ds(gb + i, L)]
                    relsp = _splat_i32(sv[0] - rbase)
                    mold = plsc.load_gather(mbuf, [relsp])
                    plsc.store_scatter(mbuf, [relsp],
                                       jnp.maximum(mold, ssp), mask=lane0)
                    return 0

                lax.fori_loop(0, CG, edge_body, 0)
                return 0

            lax.fori_loop(0, cnt_pad // CG, score_grp, 0)

            def exp_body(j, _):
                sv = ebuf[pl.ds(j * L, L)]
                rel = sels[pl.ds(j * L, L)] - rbase
                mg = plsc.load_gather(mbuf, [rel])
                ebuf[pl.ds(j * L, L)] = jnp.exp(sv - mg)
                return 0
            lax.fori_loop(0, cnt_pad // L, exp_body, 0)

            def den_body(i, _):
                ev = ebuf[pl.ds(i, L)]
                sv = sels[pl.ds(i, L)]
                relsp = _splat_i32(sv[0] - rbase)
                dold = plsc.load_gather(dbuf, [relsp])
                plsc.store_scatter(dbuf, [relsp], dold + _splat_f32(ev[0]),
                                   mask=lane0)
                return 0
            lax.fori_loop(0, cnt_pad, den_body, 0)

            def attn_body(j, _):
                ev = ebuf[pl.ds(j * L, L)]
                rel = sels[pl.ds(j * L, L)] - rbase
                dg = plsc.load_gather(dbuf, [rel])
                ebuf[pl.ds(j * L, L)] = ev / dg
                return 0
            lax.fori_loop(0, cnt_pad // L, attn_body, 0)

            def agg_grp(p, _):
                gb = p * CV
                pltpu.async_copy(v_hbm.at[seld.at[pl.ds(gb, CV)]],
                                 gbuf.at[pl.ds(0, CV)], sem).wait()

                def edge_body(i, _):
                    sv = sels[pl.ds(gb + i, L)]
                    r = sv[0] - rbase
                    av = ebuf[pl.ds(gb + i, L)]
                    asp = _splat_f32(av[0])
                    for kk in range(D // L):
                        x = x2buf[r, pl.ds(kk * L, L)]
                        x2buf[r, pl.ds(kk * L, L)] = (
                            x + asp * gbuf[i, pl.ds(kk * L, L)])
                    return 0

                lax.fori_loop(0, CV, edge_body, 0)
                return 0

            lax.fori_loop(0, cnt_pad // CV, agg_grp, 0)

            pltpu.sync_copy(x2buf.at[pl.ds(0, HALF_BASE)],
                            out_hbm.at[pl.ds(rbase, HALF_BASE)])

            @pl.when(last)
            def _():
                pltpu.sync_copy(x2buf.at[pl.ds(HALF_BASE, 8)],
                                out_hbm.at[pl.ds(rbase + HALF_BASE, 8)])

    @pl.when(cid == 0)
    def _():
        pipeline(q0, k0, v0, src0, dst0, x2_0)

    @pl.when(cid == 1)
    def _():
        pipeline(q1, k1, v1, src1, dst1, x2_1)


def _sc_sparse(qkv, edges):
    mesh = plsc.VectorSubcoreMesh(core_axis_name="c", subcore_axis_name="s",
                                  num_cores=NC, num_subcores=NS)
    f = pl.kernel(
        _sc_body,
        out_type=[jax.ShapeDtypeStruct((N, D), jnp.float32),
                  jax.ShapeDtypeStruct((N, D), jnp.float32)],
        mesh=mesh,
        compiler_params=pltpu.CompilerParams(needs_layout_passes=False),
        scratch_types=[
            pltpu.VMEM((CE,), jnp.int32),
            pltpu.VMEM((CE,), jnp.int32),
            pltpu.VMEM((CAP,), jnp.int32),
            pltpu.VMEM((CAP,), jnp.int32),
            pltpu.VMEM((CAP,), jnp.float32),
            pltpu.VMEM((CAP,), jnp.int32),
            pltpu.VMEM((CAP,), jnp.int32),
            pltpu.VMEM((CAP,), jnp.float32),
            pltpu.VMEM((CV, D), jnp.float32),
            pltpu.VMEM((TRASH + 1, D), jnp.float32),
            pltpu.VMEM((TRASH + L,), jnp.float32),
            pltpu.VMEM((TRASH + L,), jnp.float32),
            pltpu.SemaphoreType.DMA,
        ],
    )
    q0, k0, v0, q1, k1, v1 = qkv
    src0, dst0, src1, dst1 = edges
    return f(q0, k0, v0, src0, dst0, q1, k1, v1, src1, dst1)


def kernel(x, graph_indices, W1, b1, W2, b2, Wa, ba, Wo1, bo1, Wo2, bo2):
    h, a = _dense1(x, W1, b1, W2, b2, Wa, ba)
    qkv = tuple(jnp.asarray(a[:, j * D:(j + 1) * D]) for j in range(6))
    gi = graph_indices.astype(jnp.int32)
    edges = (gi[0, 0], gi[0, 1], gi[1, 0], gi[1, 1])
    x2_0, x2_1 = _sc_sparse(qkv, edges)
    cat = jnp.concatenate([h, x2_0, x2_1], axis=-1)
    return _dense2(cat, Wo1, bo1, Wo2, bo2)
